# baseline (speedup 1.0000x reference)
import functools

import jax
import jax.numpy as jnp
from jax import lax
from jax.experimental import pallas as pl
from jax.experimental.pallas import tpu as pltpu
from jax.experimental.pallas import tpu_sc as plsc

_NC = 2
_NS = 16
_NW = _NC * _NS
_LANES = 16

_CARD = 1000
_D_TOK = 16


def _sc_gather(cat_emb, xcat_flat, n_used):
    total = xcat_flat.shape[0]
    d = cat_emb.shape[1]
    per_w = total // _NW
    chunk = 80
    n_chunks = per_w // chunk
    assert n_chunks * chunk == per_w and per_w % 8 == 0

    mesh = plsc.VectorSubcoreMesh(
        core_axis_name="c", subcore_axis_name="s",
        num_cores=_NC, num_subcores=_NS,
    )

    @functools.partial(
        pl.kernel,
        out_type=jax.ShapeDtypeStruct((total, d), jnp.float32),
        mesh=mesh,
        scratch_types=[
            pltpu.VMEM((n_chunks, chunk), jnp.int32),
            pltpu.VMEM((n_chunks, chunk, d), jnp.float32),
            pltpu.SemaphoreType.DMA,
        ],
        compiler_params=pltpu.CompilerParams(use_tc_tiling_on_sc=False),
    )
    def gather_kernel(emb_hbm, idx_hbm, out_hbm, idx_v, rows_v, sem):
        wid = lax.axis_index("s") * _NC + lax.axis_index("c")
        base = wid * per_w
        for k in range(n_chunks):
            pltpu.sync_copy(idx_hbm.at[pl.ds(base + k * chunk, chunk)],
                            idx_v.at[k])
        iota = lax.broadcasted_iota(jnp.int32, (_LANES,), 0)
        for k in range(n_chunks):
            for c in range(chunk // _LANES):
                row = base + k * chunk + c * _LANES + iota
                field = lax.rem(row, n_used)
                sl = pl.ds(c * _LANES, _LANES)
                idx_v[k, sl] = idx_v[k, sl] + field * _CARD
        copies = [
            pltpu.async_copy(emb_hbm.at[idx_v.at[k]], rows_v.at[k], sem)
            for k in range(n_chunks)
        ]
        for k in range(n_chunks):
            copies[k].wait()
            pltpu.sync_copy(rows_v.at[k],
                            out_hbm.at[pl.ds(base + k * chunk, chunk)])

    return gather_kernel(cat_emb, xcat_flat)


def _tc_body(n_cat, bt, xnum_ref, g_ref, tokw_ref, tokb_ref, recw_ref,
             rlw_ref, rlb_ref, onum_ref, ocat_ref):
    f32 = jnp.float32
    xnum = xnum_ref[...]
    d_num = xnum.shape[1]

    recw = recw_ref[...]
    a = jnp.sum(tokw_ref[0:d_num, :] * recw, axis=1)
    bias13 = jnp.concatenate(
        [jnp.zeros((1, _D_TOK), f32), tokb_ref[0:d_num - 1, :]], axis=0)
    c = jnp.sum(bias13 * recw, axis=1)
    xaug = jnp.concatenate(
        [jnp.ones((bt, 1), f32), xnum[:, 0:d_num - 1]], axis=1)
    onum_ref[...] = xaug * a[None, :] + c[None, :]

    h0 = xnum[:, d_num - 1:d_num] * tokw_ref[d_num:d_num + 1, :] \
        + tokb_ref[d_num - 1:d_num, :]
    for n in range(n_cat):
        if n == 0:
            h = h0
        else:
            h = g_ref[:, n - 1, :] + tokb_ref[d_num - 1 + n:d_num + n, :]
        w = rlw_ref[n]
        out = lax.dot_general(h, w, (((1,), (1,)), ((), ())),
                              preferred_element_type=f32)
        ocat_ref[:, n, :] = out + rlb_ref[n, :][None, :]


def kernel(x_num, x_cat, tok_weight, tok_bias, cat_emb, category_offsets,
           rec_weight, rec_lin_w, rec_lin_b):
    b, d_num = x_num.shape
    n_cat = x_cat.shape[1]
    n_used = n_cat - 1
    card = rec_lin_w.shape[1]

    xcat_flat = x_cat[:, :n_used].reshape(b * n_used)
    g = _sc_gather(cat_emb, xcat_flat, n_used).reshape(b, n_used, _D_TOK)

    bt = 128
    grid = (b // bt,)
    onum, ocat = pl.pallas_call(
        functools.partial(_tc_body, n_cat, bt),
        grid=grid,
        in_specs=[
            pl.BlockSpec((bt, d_num), lambda i: (i, 0)),
            pl.BlockSpec((bt, n_used, _D_TOK), lambda i: (i, 0, 0)),
            pl.BlockSpec(tok_weight.shape, lambda i: (0, 0)),
            pl.BlockSpec(tok_bias.shape, lambda i: (0, 0)),
            pl.BlockSpec(rec_weight.shape, lambda i: (0, 0)),
            pl.BlockSpec(rec_lin_w.shape, lambda i: (0, 0, 0)),
            pl.BlockSpec(rec_lin_b.shape, lambda i: (0, 0)),
        ],
        out_specs=[
            pl.BlockSpec((bt, d_num), lambda i: (i, 0)),
            pl.BlockSpec((bt, n_cat, card), lambda i: (i, 0, 0)),
        ],
        out_shape=[
            jax.ShapeDtypeStruct((b, d_num), jnp.float32),
            jax.ShapeDtypeStruct((b, n_cat, card), jnp.float32),
        ],
    )(x_num, g, tok_weight, tok_bias, rec_weight, rec_lin_w, rec_lin_b)
    return onum, ocat


# speedup vs baseline: 1.0633x; 1.0633x over previous
import functools

import jax
import jax.numpy as jnp
from jax import lax
from jax.experimental import pallas as pl
from jax.experimental.pallas import tpu as pltpu
from jax.experimental.pallas import tpu_sc as plsc

_NC = 2
_NS = 16
_NW = _NC * _NS
_LANES = 16

_CARD = 1000
_D_TOK = 16


def _sc_gather(cat_emb, xcat_flat, b):
    total = xcat_flat.shape[0]
    d = cat_emb.shape[1]
    per_w = total // _NW
    chunk = 80
    n_chunks = per_w // chunk
    assert n_chunks * chunk == per_w and per_w % 8 == 0

    mesh = plsc.VectorSubcoreMesh(
        core_axis_name="c", subcore_axis_name="s",
        num_cores=_NC, num_subcores=_NS,
    )

    @functools.partial(
        pl.kernel,
        out_type=jax.ShapeDtypeStruct((total, d), jnp.float32),
        mesh=mesh,
        scratch_types=[
            pltpu.VMEM((n_chunks, chunk), jnp.int32),
            pltpu.VMEM((n_chunks, chunk, d), jnp.float32),
            pltpu.SemaphoreType.DMA,
        ],
        compiler_params=pltpu.CompilerParams(use_tc_tiling_on_sc=False),
    )
    def gather_kernel(emb_hbm, idx_hbm, out_hbm, idx_v, rows_v, sem):
        wid = lax.axis_index("s") * _NC + lax.axis_index("c")
        base = wid * per_w
        for k in range(n_chunks):
            pltpu.sync_copy(idx_hbm.at[pl.ds(base + k * chunk, chunk)],
                            idx_v.at[k])
        shift = b.bit_length() - 1
        iota = lax.broadcasted_iota(jnp.int32, (_LANES,), 0)
        for k in range(n_chunks):
            for c in range(chunk // _LANES):
                row = base + k * chunk + c * _LANES + iota
                field = lax.shift_right_logical(row, shift)
                sl = pl.ds(c * _LANES, _LANES)
                idx_v[k, sl] = idx_v[k, sl] + field * _CARD
        copies = [
            pltpu.async_copy(emb_hbm.at[idx_v.at[k]], rows_v.at[k], sem)
            for k in range(n_chunks)
        ]
        for k in range(n_chunks):
            copies[k].wait()
            pltpu.sync_copy(rows_v.at[k],
                            out_hbm.at[pl.ds(base + k * chunk, chunk)])

    return gather_kernel(cat_emb, xcat_flat)


def _tc_body(n_cat, bt, xnum_ref, g_ref, tokw_ref, tokb_ref, recw_ref,
             rlwt_ref, rlb_ref, onum_ref, ocat_ref):
    f32 = jnp.float32
    xnum = xnum_ref[...]
    d_num = xnum.shape[1]

    recw = recw_ref[...]
    a = jnp.sum(tokw_ref[0:d_num, :] * recw, axis=1)
    bias13 = jnp.concatenate(
        [jnp.zeros((1, _D_TOK), f32), tokb_ref[0:d_num - 1, :]], axis=0)
    c = jnp.sum(bias13 * recw, axis=1)
    rows = lax.broadcasted_iota(jnp.int32, (d_num, d_num), 0)
    cols = lax.broadcasted_iota(jnp.int32, (d_num, d_num), 1)
    m = jnp.where(cols == rows + 1, a[None, :], jnp.zeros((), f32))
    c2 = c + jnp.where(
        lax.broadcasted_iota(jnp.int32, (d_num,), 0) == 0, a[0], 0.0)
    onum_ref[...] = lax.dot_general(
        xnum, m, (((1,), (0,)), ((), ())),
        preferred_element_type=f32) + c2[None, :]

    h0 = xnum[:, d_num - 1:d_num] * tokw_ref[d_num:d_num + 1, :] \
        + tokb_ref[d_num - 1:d_num, :]
    for n in range(n_cat):
        if n == 0:
            h = h0
        else:
            h = g_ref[n - 1] + tokb_ref[d_num - 1 + n:d_num + n, :]
        wt = rlwt_ref[n]
        out = lax.dot_general(h, wt, (((1,), (0,)), ((), ())),
                              preferred_element_type=f32)
        ocat_ref[:, n, :] = out + rlb_ref[n, :][None, :]


def kernel(x_num, x_cat, tok_weight, tok_bias, cat_emb, category_offsets,
           rec_weight, rec_lin_w, rec_lin_b):
    b, d_num = x_num.shape
    n_cat = x_cat.shape[1]
    n_used = n_cat - 1
    card = rec_lin_w.shape[1]

    xcat_flat = x_cat[:, :n_used].T.reshape(n_used * b)
    g = _sc_gather(cat_emb, xcat_flat, b).reshape(n_used, b, _D_TOK)
    rlwt = rec_lin_w.transpose(0, 2, 1)

    bt = 128
    grid = (b // bt,)
    onum, ocat = pl.pallas_call(
        functools.partial(_tc_body, n_cat, bt),
        grid=grid,
        in_specs=[
            pl.BlockSpec((bt, d_num), lambda i: (i, 0)),
            pl.BlockSpec((n_used, bt, _D_TOK), lambda i: (0, i, 0)),
            pl.BlockSpec(tok_weight.shape, lambda i: (0, 0)),
            pl.BlockSpec(tok_bias.shape, lambda i: (0, 0)),
            pl.BlockSpec(rec_weight.shape, lambda i: (0, 0)),
            pl.BlockSpec(rlwt.shape, lambda i: (0, 0, 0)),
            pl.BlockSpec(rec_lin_b.shape, lambda i: (0, 0)),
        ],
        out_specs=[
            pl.BlockSpec((bt, d_num), lambda i: (i, 0)),
            pl.BlockSpec((bt, n_cat, card), lambda i: (i, 0, 0)),
        ],
        out_shape=[
            jax.ShapeDtypeStruct((b, d_num), jnp.float32),
            jax.ShapeDtypeStruct((b, n_cat, card), jnp.float32),
        ],
    )(x_num, g, tok_weight, tok_bias, rec_weight, rlwt, rec_lin_b)
    return onum, ocat


# speedup vs baseline: 1.0641x; 1.0008x over previous
import functools

import jax
import jax.numpy as jnp
from jax import lax
from jax.experimental import pallas as pl
from jax.experimental.pallas import tpu as pltpu
from jax.experimental.pallas import tpu_sc as plsc

_NC = 2
_NS = 16
_NW = _NC * _NS
_LANES = 16

_CARD = 1000
_D_TOK = 16


def _sc_gather(cat_emb, xcat_flat, b):
    total = xcat_flat.shape[0]
    d = cat_emb.shape[1]
    per_w = total // _NW
    chunk = 80
    n_chunks = per_w // chunk
    assert n_chunks * chunk == per_w and per_w % 8 == 0

    mesh = plsc.VectorSubcoreMesh(
        core_axis_name="c", subcore_axis_name="s",
        num_cores=_NC, num_subcores=_NS,
    )

    @functools.partial(
        pl.kernel,
        out_type=jax.ShapeDtypeStruct((total, d), jnp.float32),
        mesh=mesh,
        scratch_types=[
            pltpu.VMEM((n_chunks, chunk), jnp.int32),
            pltpu.VMEM((n_chunks, chunk, d), jnp.float32),
            pltpu.SemaphoreType.DMA,
        ],
        compiler_params=pltpu.CompilerParams(use_tc_tiling_on_sc=False),
    )
    def gather_kernel(emb_hbm, idx_hbm, out_hbm, idx_v, rows_v, sem):
        wid = lax.axis_index("s") * _NC + lax.axis_index("c")
        base = wid * per_w
        for k in range(n_chunks):
            pltpu.sync_copy(idx_hbm.at[pl.ds(base + k * chunk, chunk)],
                            idx_v.at[k])
        shift = b.bit_length() - 1
        iota = lax.broadcasted_iota(jnp.int32, (_LANES,), 0)
        for k in range(n_chunks):
            for c in range(chunk // _LANES):
                row = base + k * chunk + c * _LANES + iota
                field = lax.shift_right_logical(row, shift)
                sl = pl.ds(c * _LANES, _LANES)
                idx_v[k, sl] = idx_v[k, sl] + field * _CARD
        copies = [
            pltpu.async_copy(emb_hbm.at[idx_v.at[k]], rows_v.at[k], sem)
            for k in range(n_chunks)
        ]
        for k in range(n_chunks):
            copies[k].wait()
            pltpu.sync_copy(rows_v.at[k],
                            out_hbm.at[pl.ds(base + k * chunk, chunk)])

    return gather_kernel(cat_emb, xcat_flat)


def _tc_body(n_cat, bt, n_dma, xnum_ref, g_ref, tokw_ref, tokb_ref, recw_ref,
             rlwt_ref, rlb_ref, onum_ref, ocat_hbm, buf0, buf1, sems):
    f32 = jnp.float32
    xnum = xnum_ref[...]
    d_num = xnum.shape[1]
    i = pl.program_id(0)
    nstep = pl.num_programs(0)
    card = ocat_hbm.shape[2]
    rc = bt // n_dma

    recw = recw_ref[...]
    a = jnp.sum(tokw_ref[0:d_num, :] * recw, axis=1)
    bias13 = jnp.concatenate(
        [jnp.zeros((1, _D_TOK), f32), tokb_ref[0:d_num - 1, :]], axis=0)
    c = jnp.sum(bias13 * recw, axis=1)
    rows = lax.broadcasted_iota(jnp.int32, (d_num, d_num), 0)
    cols = lax.broadcasted_iota(jnp.int32, (d_num, d_num), 1)
    m = jnp.where(cols == rows + 1, a[None, :], jnp.zeros((), f32))
    c2 = c + jnp.where(
        lax.broadcasted_iota(jnp.int32, (d_num,), 0) == 0, a[0], 0.0)
    onum_ref[...] = lax.dot_general(
        xnum, m, (((1,), (0,)), ((), ())),
        preferred_element_type=f32) + c2[None, :]

    h0 = xnum[:, d_num - 1:d_num] * tokw_ref[d_num:d_num + 1, :] \
        + tokb_ref[d_num - 1:d_num, :]

    def compute_into(buf):
        for n in range(n_cat):
            if n == 0:
                h = h0
            else:
                h = g_ref[n - 1] + tokb_ref[d_num - 1 + n:d_num + n, :]
            wt = rlwt_ref[n]
            out = lax.dot_general(h, wt, (((1,), (0,)), ((), ())),
                                  preferred_element_type=f32)
            buf[:, n, :] = out + rlb_ref[n, :][None, :]

    def dma(buf, slot, step):
        return [
            pltpu.make_async_copy(
                buf.at[pl.ds(s * rc, rc)],
                ocat_hbm.at[pl.ds(step * bt + s * rc, rc)],
                sems.at[slot, s])
            for s in range(n_dma)
        ]

    slot = lax.rem(i, 2)
    for k, buf in ((0, buf0), (1, buf1)):
        @pl.when(slot == k)
        def _(buf=buf):
            compute_into(buf)

    for k, buf in ((0, buf0), (1, buf1)):
        @pl.when((i >= 1) & (slot == 1 - k))
        def _(k=k, buf=buf):
            for c in dma(buf, k, i - 1):
                c.wait()

    for k, buf in ((0, buf0), (1, buf1)):
        @pl.when(slot == k)
        def _(k=k, buf=buf):
            for c in dma(buf, k, i):
                c.start()

    for k, buf in ((0, buf0), (1, buf1)):
        @pl.when((i == nstep - 1) & (slot == k))
        def _(k=k, buf=buf):
            for c in dma(buf, k, i):
                c.wait()


def kernel(x_num, x_cat, tok_weight, tok_bias, cat_emb, category_offsets,
           rec_weight, rec_lin_w, rec_lin_b):
    b, d_num = x_num.shape
    n_cat = x_cat.shape[1]
    n_used = n_cat - 1
    card = rec_lin_w.shape[1]

    xcat_flat = x_cat[:, :n_used].T.reshape(n_used * b)
    g = _sc_gather(cat_emb, xcat_flat, b).reshape(n_used, b, _D_TOK)
    rlwt = rec_lin_w.transpose(0, 2, 1)

    bt = 128
    n_dma = 4
    grid = (b // bt,)
    onum, ocat = pl.pallas_call(
        functools.partial(_tc_body, n_cat, bt, n_dma),
        grid=grid,
        in_specs=[
            pl.BlockSpec((bt, d_num), lambda i: (i, 0)),
            pl.BlockSpec((n_used, bt, _D_TOK), lambda i: (0, i, 0)),
            pl.BlockSpec(tok_weight.shape, lambda i: (0, 0)),
            pl.BlockSpec(tok_bias.shape, lambda i: (0, 0)),
            pl.BlockSpec(rec_weight.shape, lambda i: (0, 0)),
            pl.BlockSpec(rlwt.shape, lambda i: (0, 0, 0)),
            pl.BlockSpec(rec_lin_b.shape, lambda i: (0, 0)),
        ],
        out_specs=[
            pl.BlockSpec((bt, d_num), lambda i: (i, 0)),
            pl.BlockSpec(memory_space=pltpu.MemorySpace.HBM),
        ],
        out_shape=[
            jax.ShapeDtypeStruct((b, d_num), jnp.float32),
            jax.ShapeDtypeStruct((b, n_cat, card), jnp.float32),
        ],
        scratch_shapes=[
            pltpu.VMEM((bt, n_cat, card), jnp.float32),
            pltpu.VMEM((bt, n_cat, card), jnp.float32),
            pltpu.SemaphoreType.DMA((2, n_dma)),
        ],
    )(x_num, g, tok_weight, tok_bias, rec_weight, rlwt, rec_lin_b)
    return onum, ocat
